# baseline (speedup 1.0000x reference)
import functools

import numpy as np
import jax
import jax.numpy as jnp
from jax import lax
from jax.experimental import pallas as pl
from jax.experimental.pallas import tpu as pltpu
from jax.experimental.pallas import tpu_sc as plsc

_CHUNK_FRACS = (0.625, 0.25, 0.125)
_WINDOW = 256
_BM_MLP = 1024
_BM_LOSS = 1024
_N_OUT = 24


def _sc_gather(table, indices, window):
    n = indices.shape[0]
    d = table.shape[1]
    indices = indices.reshape(1, n)
    mesh = plsc.VectorSubcoreMesh(core_axis_name="core", subcore_axis_name="subcore")

    @functools.partial(
        pl.kernel,
        out_type=jax.ShapeDtypeStruct((n, d), table.dtype),
        mesh=mesh,
    )
    def gather_kernel(tbl_hbm, idx_hbm, out_hbm):
        def body(idx_vmem, out_vmem):
            pltpu.sync_copy(tbl_hbm.at[idx_vmem.at[0]], out_vmem)

        pltpu.emit_pipeline(
            body,
            grid=(n // window,),
            in_specs=[pl.BlockSpec((1, window), index_map=lambda i: (0, i))],
            out_specs=[pl.BlockSpec((window, d), index_map=lambda i: (i, 0))],
            core_axis_name=("core", "subcore"),
            dimension_semantics=(pltpu.PARALLEL,),
        )(idx_hbm, out_hbm)

    return gather_kernel(table, indices)


def _mlp_body(x_ref, w1_ref, b1_ref, w2_ref, b2_ref, o_ref):
    h = jnp.dot(
        x_ref[...].astype(jnp.bfloat16), w1_ref[...],
        preferred_element_type=jnp.float32,
    )
    h = jnp.maximum(h + b1_ref[...], 0.0)
    o = jnp.dot(
        h.astype(jnp.bfloat16), w2_ref[...],
        preferred_element_type=jnp.float32,
    )
    o_ref[...] = jnp.maximum(o + b2_ref[...], 0.0).astype(jnp.bfloat16)


def _tc_mlp(x, w1, b1, w2, b2, bm):
    b, d_in = x.shape
    h1 = w1.shape[1]
    d_out = w2.shape[1]
    return pl.pallas_call(
        _mlp_body,
        grid=(b // bm,),
        in_specs=[
            pl.BlockSpec((bm, d_in), lambda i: (i, 0)),
            pl.BlockSpec((d_in, h1), lambda i: (0, 0)),
            pl.BlockSpec((1, h1), lambda i: (0, 0)),
            pl.BlockSpec((h1, d_out), lambda i: (0, 0)),
            pl.BlockSpec((1, d_out), lambda i: (0, 0)),
        ],
        out_specs=pl.BlockSpec((bm, d_out), lambda i: (i, 0)),
        out_shape=jax.ShapeDtypeStruct((b, d_out), jnp.bfloat16),
    )(x, w1.astype(jnp.bfloat16), b1.reshape(1, h1),
      w2.astype(jnp.bfloat16), b2.reshape(1, d_out))


def _loss_body(n_neg, u_ref, neg_ref, mask_ref, o_ref):
    i = pl.program_id(0)
    u = u_ref[...]
    bm = u.shape[0]
    prods = [(-u) * neg_ref[0].astype(jnp.bfloat16)]
    prods += [neg_ref[nn].astype(jnp.bfloat16) * u for nn in range(1, n_neg + 1)]
    m = jnp.concatenate(prods, axis=1)
    ns = lax.dot_general(
        m, mask_ref[...], (((1,), (0,)), ((), ())),
        preferred_element_type=jnp.float32,
    )
    ls = jnp.log1p(jnp.exp(jnp.clip(ns, -10.0, 10.0)))
    pad_term = (_N_OUT - (n_neg + 1)) * bm * float(np.log(2.0))
    part = (jnp.sum(ls) - pad_term)[None, None]

    @pl.when(i == 0)
    def _():
        o_ref[...] = jnp.zeros_like(o_ref)

    o_ref[...] += part


def _tc_loss_chunk(u, rows3, mask, row_base, chunk_b, n_neg, bm):
    d = u.shape[1]
    base = row_base // bm
    return pl.pallas_call(
        functools.partial(_loss_body, n_neg),
        grid=(chunk_b // bm,),
        in_specs=[
            pl.BlockSpec((bm, d), lambda i: (base + i, 0)),
            pl.BlockSpec((n_neg + 1, bm, d), lambda i: (0, i, 0)),
            pl.BlockSpec(((n_neg + 1) * d, _N_OUT), lambda i: (0, 0)),
        ],
        out_specs=pl.BlockSpec((1, 1), lambda i: (0, 0)),
        out_shape=jax.ShapeDtypeStruct((1, 1), jnp.float32),
    )(u, rows3, mask)


def kernel(pos_input, pos_item, neg_item, input_emb, item_emb, W1, b1, W2, b2):
    b = pos_input.shape[0]
    n_neg = neg_item.shape[1]
    d = item_emb.shape[1]

    mask = (
        lax.broadcasted_iota(jnp.int32, ((n_neg + 1) * d, _N_OUT), 0) // d
        == lax.broadcasted_iota(jnp.int32, ((n_neg + 1) * d, _N_OUT), 1)
    ).astype(jnp.bfloat16)

    x = _sc_gather(input_emb, pos_input.astype(jnp.int32), window=_WINDOW)
    u = _tc_mlp(x, W1, b1, W2, b2, bm=_BM_MLP)

    neg_i32 = neg_item.astype(jnp.int32)
    pos_i32 = pos_item.astype(jnp.int32)
    parts = []
    row = 0
    for frac in _CHUNK_FRACS:
        cb = int(b * frac)
        idx_c = jnp.concatenate(
            [pos_i32[row:row + cb], neg_i32[row:row + cb].transpose(1, 0).reshape(-1)]
        )
        g = _sc_gather(item_emb, idx_c, window=_WINDOW)
        rows3 = g.reshape(n_neg + 1, cb, d)
        parts.append(
            _tc_loss_chunk(u, rows3, mask, row, cb, n_neg, bm=_BM_LOSS)
        )
        row += cb

    total = sum(p[0, 0] for p in parts)
    return (total / b).astype(jnp.float32)


# speedup vs baseline: 1.0110x; 1.0110x over previous
import functools

import numpy as np
import jax
import jax.numpy as jnp
from jax import lax
from jax.experimental import pallas as pl
from jax.experimental.pallas import tpu as pltpu
from jax.experimental.pallas import tpu_sc as plsc

_CHUNK_FRACS = (0.5, 0.25, 0.25)
_WINDOW = 256
_BM_MLP = 1024
_BM_LOSS = 1024
_N_OUT = 24


def _sc_gather(table, indices, window):
    n = indices.shape[0]
    d = table.shape[1]
    indices = indices.reshape(1, n)
    mesh = plsc.VectorSubcoreMesh(core_axis_name="core", subcore_axis_name="subcore")

    @functools.partial(
        pl.kernel,
        out_type=jax.ShapeDtypeStruct((n, d), table.dtype),
        mesh=mesh,
    )
    def gather_kernel(tbl_hbm, idx_hbm, out_hbm):
        def body(idx_vmem, out_vmem):
            pltpu.sync_copy(tbl_hbm.at[idx_vmem.at[0]], out_vmem)

        pltpu.emit_pipeline(
            body,
            grid=(n // window,),
            in_specs=[pl.BlockSpec((1, window), index_map=lambda i: (0, i))],
            out_specs=[pl.BlockSpec((window, d), index_map=lambda i: (i, 0))],
            core_axis_name=("core", "subcore"),
            dimension_semantics=(pltpu.PARALLEL,),
        )(idx_hbm, out_hbm)

    return gather_kernel(table, indices)


def _mlp_body(x_ref, w1_ref, b1_ref, w2_ref, b2_ref, o_ref):
    h = jnp.dot(
        x_ref[...].astype(jnp.bfloat16), w1_ref[...],
        preferred_element_type=jnp.float32,
    )
    h = jnp.maximum(h + b1_ref[...], 0.0)
    o = jnp.dot(
        h.astype(jnp.bfloat16), w2_ref[...],
        preferred_element_type=jnp.float32,
    )
    o_ref[...] = jnp.maximum(o + b2_ref[...], 0.0).astype(jnp.bfloat16)


def _tc_mlp(x, w1, b1, w2, b2, bm):
    b, d_in = x.shape
    h1 = w1.shape[1]
    d_out = w2.shape[1]
    return pl.pallas_call(
        _mlp_body,
        grid=(b // bm,),
        in_specs=[
            pl.BlockSpec((bm, d_in), lambda i: (i, 0)),
            pl.BlockSpec((d_in, h1), lambda i: (0, 0)),
            pl.BlockSpec((1, h1), lambda i: (0, 0)),
            pl.BlockSpec((h1, d_out), lambda i: (0, 0)),
            pl.BlockSpec((1, d_out), lambda i: (0, 0)),
        ],
        out_specs=pl.BlockSpec((bm, d_out), lambda i: (i, 0)),
        out_shape=jax.ShapeDtypeStruct((b, d_out), jnp.bfloat16),
    )(x, w1.astype(jnp.bfloat16), b1.reshape(1, h1),
      w2.astype(jnp.bfloat16), b2.reshape(1, d_out))


def _loss_body(n_neg, u_ref, neg_ref, mask_ref, o_ref):
    i = pl.program_id(0)
    u = u_ref[...]
    bm = u.shape[0]
    prods = [(-u) * neg_ref[0].astype(jnp.bfloat16)]
    prods += [neg_ref[nn].astype(jnp.bfloat16) * u for nn in range(1, n_neg + 1)]
    m = jnp.concatenate(prods, axis=1)
    ns = lax.dot_general(
        m, mask_ref[...], (((1,), (0,)), ((), ())),
        preferred_element_type=jnp.float32,
    )
    ls = jnp.log1p(jnp.exp(jnp.clip(ns, -10.0, 10.0)))
    pad_term = (_N_OUT - (n_neg + 1)) * bm * float(np.log(2.0))
    part = (jnp.sum(ls) - pad_term)[None, None]

    @pl.when(i == 0)
    def _():
        o_ref[...] = jnp.zeros_like(o_ref)

    o_ref[...] += part


def _tc_loss_chunk(u, rows3, mask, row_base, chunk_b, n_neg, bm):
    d = u.shape[1]
    base = row_base // bm
    return pl.pallas_call(
        functools.partial(_loss_body, n_neg),
        grid=(chunk_b // bm,),
        in_specs=[
            pl.BlockSpec((bm, d), lambda i: (base + i, 0)),
            pl.BlockSpec((n_neg + 1, bm, d), lambda i: (0, i, 0)),
            pl.BlockSpec(((n_neg + 1) * d, _N_OUT), lambda i: (0, 0)),
        ],
        out_specs=pl.BlockSpec((1, 1), lambda i: (0, 0)),
        out_shape=jax.ShapeDtypeStruct((1, 1), jnp.float32),
    )(u, rows3, mask)


def kernel(pos_input, pos_item, neg_item, input_emb, item_emb, W1, b1, W2, b2):
    b = pos_input.shape[0]
    n_neg = neg_item.shape[1]
    d = item_emb.shape[1]

    mask = (
        lax.broadcasted_iota(jnp.int32, ((n_neg + 1) * d, _N_OUT), 0) // d
        == lax.broadcasted_iota(jnp.int32, ((n_neg + 1) * d, _N_OUT), 1)
    ).astype(jnp.bfloat16)

    x = _sc_gather(input_emb, pos_input.astype(jnp.int32), window=_WINDOW)
    u = _tc_mlp(x, W1, b1, W2, b2, bm=_BM_MLP)

    neg_i32 = neg_item.astype(jnp.int32)
    pos_i32 = pos_item.astype(jnp.int32)
    parts = []
    row = 0
    for frac in _CHUNK_FRACS:
        cb = int(b * frac)
        idx_c = jnp.concatenate(
            [pos_i32[row:row + cb], neg_i32[row:row + cb].transpose(1, 0).reshape(-1)]
        )
        g = _sc_gather(item_emb, idx_c, window=_WINDOW)
        rows3 = g.reshape(n_neg + 1, cb, d)
        parts.append(
            _tc_loss_chunk(u, rows3, mask, row, cb, n_neg, bm=_BM_LOSS)
        )
        row += cb

    total = sum(p[0, 0] for p in parts)
    return (total / b).astype(jnp.float32)
